# baseline (speedup 1.0000x reference)
import functools

import jax
import jax.numpy as jnp
from jax import lax
from jax.experimental import pallas as pl
from jax.experimental.pallas import tpu as pltpu
from jax.experimental.pallas import tpu_sc as plsc

_N_CLASSES = 1000000
_D = 16
_B = 16384

_info = plsc.get_sparse_core_info()
_NC, _NS, _L = _info.num_cores, _info.num_subcores, _info.num_lanes
_NW = _NC * _NS
_BPW = _B // _NW
_NCHUNK = _BPW // _L
_NSUB = _BPW // 128
_PAD_POS = _NSUB * 128
_NBANK = 3
_HALF = _B // _NW // 2 + 16

_mesh = plsc.VectorSubcoreMesh(core_axis_name="c", subcore_axis_name="s")


@functools.partial(
    pl.kernel,
    mesh=_mesh,
    out_type=jax.ShapeDtypeStruct((_D, _B), jnp.float32),
    scratch_types=[
        pltpu.VMEM((_BPW,), jnp.int32),
        pltpu.VMEM((_BPW,), jnp.int32),
        pltpu.VMEM((_NSUB + 1, _D, 128), jnp.float32),
        pltpu.VMEM((_L,), jnp.float32),
        pltpu.SMEM((2 * (_BPW // 2 + _L),), jnp.int32),
        pltpu.VMEM_SHARED((_NS, _NBANK * _L, _D, 128), jnp.float32),
        pltpu.SemaphoreType.DMA,
        pltpu.SemaphoreType.DMA,
        pltpu.SemaphoreType.DMA,
        pltpu.SemaphoreType.DMA,
        pltpu.SemaphoreType.DMA,
        pltpu.SemaphoreType.DMA,
    ],
)
def _emb_lookup(c_hbm, m_hbm, tt_hbm, nrow_hbm, out_hbm,
                cidx_v, m_v, out_v, nrow_v, list_s, slots_v,
                bsem0, bsem1, bsem2, csem0, csem1, csem2):
    bsems = (bsem0, bsem1, bsem2)
    csems = (csem0, csem1, csem2)
    sid = lax.axis_index("s")
    wid = sid * _NC + lax.axis_index("c")
    base = pl.multiple_of(wid * _BPW, 128)
    pltpu.async_copy(c_hbm.at[pl.ds(base, _BPW)], cidx_v, csem0)
    pltpu.async_copy(m_hbm.at[pl.ds(base, _BPW)], m_v, csem0)
    pltpu.async_copy(nrow_hbm, nrow_v, csem0)
    pltpu.make_async_copy(c_hbm.at[pl.ds(base, _BPW)], cidx_v, csem0).wait()
    pltpu.make_async_copy(m_hbm.at[pl.ds(base, _BPW)], m_v, csem0).wait()
    pltpu.make_async_copy(nrow_hbm, nrow_v, csem0).wait()

    nr = nrow_v[pl.ds(0, _L)]
    rows = [jnp.broadcast_to(nr[ch], (_L,)) for ch in range(_D)]
    for g in range(_NSUB):
        for ch in range(_D):
            for j in range(128 // _L):
                out_v[g, ch, pl.ds(j * _L, _L)] = rows[ch]

    iota = jnp.arange(_L, dtype=jnp.int32)

    def compact(p, cnts):
        cnt_a, cnt_b = cnts
        cv = cidx_v[pl.ds(p * _L, _L)]
        mv = m_v[pl.ds(p * _L, _L)]
        packed = cv + (iota + p * _L) * 1048576
        for l in range(_L // 2):
            list_s[cnt_a + 0] = packed[l]
            cnt_a = cnt_a + jnp.where(mv[l] == 0, 1, 0).astype(jnp.int32)
            list_s[cnt_b + _HALF] = packed[l + 8]
            cnt_b = cnt_b + jnp.where(mv[l + 8] == 0, 1, 0).astype(jnp.int32)
        return (cnt_a, cnt_b)

    cnt_a, cnt_b = lax.fori_loop(0, _NCHUNK, compact,
                                 (jnp.int32(0), jnp.int32(0)))

    n_ga = (cnt_a + _L - 1) >> 4
    n_gb = (cnt_b + _L - 1) >> 4
    n_grp = n_ga + n_gb
    pad_word = _PAD_POS * 1048576 + base

    def pad(k, carry):
        list_s[k] = pad_word
        return carry

    lax.fori_loop(cnt_a, n_ga << 4, pad, jnp.int32(0))

    def pad_b(k, carry):
        list_s[k + _HALF] = pad_word
        return carry

    lax.fori_loop(cnt_b, n_gb << 4, pad_b, jnp.int32(0))

    def gbase(g):
        return jnp.where(g < n_ga, g << 4, _HALF + ((g - n_ga) << 4))

    my_slots = slots_v.at[sid]

    def issue(g, q):
        gb = gbase(g)
        for l in range(_L):
            w = list_s[gb + l]
            blk = pl.multiple_of(((w & 0xFFFFF) >> 7) << 7, 128)
            pltpu.async_copy(tt_hbm.at[:, pl.ds(blk, 128)],
                             my_slots.at[q * _L + l], bsems[q])

    def drain_blocks(q):
        for l in range(_L):
            pltpu.make_async_copy(tt_hbm.at[:, pl.ds(0, 128)],
                                  my_slots.at[l], bsems[q]).wait()

    def issue_cols(g, q):
        gb = gbase(g)
        for l in range(_L):
            w = list_s[gb + l]
            col = w & 127
            pos = w >> 20
            pltpu.async_copy(
                my_slots.at[q * _L + l].at[:, pl.ds(col, 1)],
                out_v.at[pos >> 7].at[:, pl.ds(pos & 127, 1)], csems[q])

    def drain_cols(q):
        for l in range(_L):
            pltpu.make_async_copy(
                my_slots.at[l].at[:, pl.ds(0, 1)],
                out_v.at[0].at[:, pl.ds(0, 1)], csems[q]).wait()

    def for_parity(gdyn, fn):
        for q in range(_NBANK):
            @pl.when(lax.rem(gdyn, _NBANK) == q)
            def _():
                fn(q)

    @pl.when(n_grp > 0)
    def _prologue():
        issue(jnp.int32(0), 0)

    def body(g, carry):
        @pl.when(g >= 2)
        def _():
            for_parity(g + 1, drain_cols)

        @pl.when(g + 1 < n_grp)
        def _():
            for_parity(g + 1, lambda q: issue(g + 1, q))

        for_parity(g, drain_blocks)
        for_parity(g, lambda q: issue_cols(g, q))
        return carry

    lax.fori_loop(0, n_grp, body, jnp.int32(0))

    @pl.when(n_grp > 0)
    def _epi1():
        for_parity(n_grp - 1, drain_cols)

    @pl.when(n_grp > 1)
    def _epi2():
        for_parity(n_grp - 2, drain_cols)

    for j in range(_NSUB):
        pltpu.sync_copy(out_v.at[j],
                        out_hbm.at[:, pl.ds(base + j * 128, 128)])


def kernel(c, drop_mask, table):
    out_t = _emb_lookup(c.astype(jnp.int32), drop_mask.astype(jnp.int32),
                        table.T, table[_N_CLASSES])
    return out_t.T


# speedup vs baseline: 1.0266x; 1.0266x over previous
import functools

import jax
import jax.numpy as jnp
from jax import lax
from jax.experimental import pallas as pl
from jax.experimental.pallas import tpu as pltpu
from jax.experimental.pallas import tpu_sc as plsc

_N_CLASSES = 1000000
_D = 16
_B = 16384

_info = plsc.get_sparse_core_info()
_NC, _NS, _L = _info.num_cores, _info.num_subcores, _info.num_lanes
_NW = _NC * _NS
_BPW = _B // _NW
_NCHUNK = _BPW // _L
_NSUB = _BPW // 128
_PAD_POS = _NSUB * 128
_NBANK = 3

_mesh = plsc.VectorSubcoreMesh(core_axis_name="c", subcore_axis_name="s")


@functools.partial(
    pl.kernel,
    mesh=_mesh,
    out_type=jax.ShapeDtypeStruct((_D, _B), jnp.float32),
    scratch_types=[
        pltpu.VMEM((_BPW,), jnp.int32),
        pltpu.VMEM((_BPW,), jnp.int32),
        pltpu.VMEM((_NSUB + 1, _D, 128), jnp.float32),
        pltpu.VMEM((_L,), jnp.float32),
        pltpu.SMEM((_BPW + _L,), jnp.int32),
        pltpu.VMEM_SHARED((_NS, _NBANK * _L, _D, 128), jnp.float32),
        pltpu.SemaphoreType.DMA,
        pltpu.SemaphoreType.DMA,
        pltpu.SemaphoreType.DMA,
        pltpu.SemaphoreType.DMA,
        pltpu.SemaphoreType.DMA,
        pltpu.SemaphoreType.DMA,
    ],
)
def _emb_lookup(c_hbm, m_hbm, tt_hbm, nrow_hbm, out_hbm,
                cidx_v, m_v, out_v, nrow_v, list_s, slots_v,
                bsem0, bsem1, bsem2, csem0, csem1, csem2):
    bsems = (bsem0, bsem1, bsem2)
    csems = (csem0, csem1, csem2)
    sid = lax.axis_index("s")
    wid = sid * _NC + lax.axis_index("c")
    base = pl.multiple_of(wid * _BPW, 128)
    pltpu.async_copy(c_hbm.at[pl.ds(base, _BPW)], cidx_v, csem0)
    pltpu.async_copy(m_hbm.at[pl.ds(base, _BPW)], m_v, csem0)
    pltpu.async_copy(nrow_hbm, nrow_v, csem0)
    pltpu.make_async_copy(c_hbm.at[pl.ds(base, _BPW)], cidx_v, csem0).wait()
    pltpu.make_async_copy(m_hbm.at[pl.ds(base, _BPW)], m_v, csem0).wait()
    pltpu.make_async_copy(nrow_hbm, nrow_v, csem0).wait()

    nr = nrow_v[pl.ds(0, _L)]
    rows = [jnp.broadcast_to(nr[ch], (_L,)) for ch in range(_D)]
    for g in range(_NSUB):
        for ch in range(_D):
            for j in range(128 // _L):
                out_v[g, ch, pl.ds(j * _L, _L)] = rows[ch]

    iota = jnp.arange(_L, dtype=jnp.int32)

    def compact(p, cnt):
        cv = cidx_v[pl.ds(p * _L, _L)]
        mv = m_v[pl.ds(p * _L, _L)]
        packed = cv + (iota + p * _L) * 1048576
        for l in range(_L):
            list_s[cnt + 0] = packed[l]
            cnt = cnt + jnp.where(mv[l] == 0, 1, 0).astype(jnp.int32)
        return cnt

    n_valid = lax.fori_loop(0, _NCHUNK, compact, jnp.int32(0))

    n_grp = (n_valid + _L - 1) >> 4
    n_pad = n_grp << 4
    pad_word = _PAD_POS * 1048576 + base

    def pad(k, carry):
        list_s[k] = pad_word
        return carry

    lax.fori_loop(n_valid, n_pad, pad, jnp.int32(0))

    my_slots = slots_v.at[sid]

    def issue(g, q):
        for l in range(_L):
            w = list_s[g * _L + l]
            blk = pl.multiple_of(((w & 0xFFFFF) >> 7) << 7, 128)
            pltpu.async_copy(tt_hbm.at[:, pl.ds(blk, 128)],
                             my_slots.at[q * _L + l], bsems[q])

    def drain_blocks(q):
        for l in range(_L):
            pltpu.make_async_copy(tt_hbm.at[:, pl.ds(0, 128)],
                                  my_slots.at[l], bsems[q]).wait()

    def issue_cols(g, q):
        for l in range(_L):
            w = list_s[g * _L + l]
            col = w & 127
            pos = w >> 20
            pltpu.async_copy(
                my_slots.at[q * _L + l].at[:, pl.ds(col, 1)],
                out_v.at[pos >> 7].at[:, pl.ds(pos & 127, 1)], csems[q])

    def drain_cols(q):
        for l in range(_L):
            pltpu.make_async_copy(
                my_slots.at[l].at[:, pl.ds(0, 1)],
                out_v.at[0].at[:, pl.ds(0, 1)], csems[q]).wait()

    def for_parity(gdyn, fn):
        for q in range(_NBANK):
            @pl.when(lax.rem(gdyn, _NBANK) == q)
            def _():
                fn(q)

    @pl.when(n_grp > 0)
    def _prologue():
        issue(jnp.int32(0), 0)

    def body(g, carry):
        @pl.when(g >= 2)
        def _():
            for_parity(g + 1, drain_cols)

        @pl.when(g + 1 < n_grp)
        def _():
            for_parity(g + 1, lambda q: issue(g + 1, q))

        for_parity(g, drain_blocks)
        for_parity(g, lambda q: issue_cols(g, q))
        return carry

    lax.fori_loop(0, n_grp, body, jnp.int32(0))

    @pl.when(n_grp > 0)
    def _epi1():
        for_parity(n_grp - 1, drain_cols)

    @pl.when(n_grp > 1)
    def _epi2():
        for_parity(n_grp - 2, drain_cols)

    for j in range(_NSUB):
        pltpu.sync_copy(out_v.at[j],
                        out_hbm.at[:, pl.ds(base + j * 128, 128)])


def kernel(c, drop_mask, table):
    out_t = _emb_lookup(c.astype(jnp.int32), drop_mask.astype(jnp.int32),
                        table.T, table[_N_CLASSES])
    return out_t.T
